# baseline (speedup 1.0000x reference)
import functools

import jax
import jax.numpy as jnp
import numpy as np
from jax import lax
from jax.experimental import pallas as pl
from jax.experimental.pallas import tpu as pltpu
from jax.experimental.pallas import tpu_sc as plsc


def _disabled_tofs(tof_count, min_c, max_c, neighbor_p, seed=0):
    rng = np.random.default_rng(seed)
    count = int(rng.integers(min_c, max_c + 1))
    tof_list = rng.permutation(tof_count)
    first = int(rng.integers(1, tof_count))
    disabled = [first]
    tof_list = tof_list[tof_list != first]
    for _ in range(count - 1):
        r = float(rng.random())
        if r < neighbor_p:
            if r < neighbor_p / 2.0:
                offsets = (1, -1)
            else:
                offsets = (tof_count // 2, -(tof_count // 2))
            appended = False
            for d in list(disabled):
                for off in offsets:
                    cand = d + off
                    if cand in tof_list:
                        tof_list = tof_list[tof_list != cand]
                        disabled.append(int(cand))
                        appended = True
                        break
                if appended:
                    break
            if not appended:
                new = int(tof_list[0])
                tof_list = tof_list[tof_list != new]
                disabled.append(new)
        else:
            new = int(tof_list[0])
            tof_list = tof_list[tof_list != new]
            disabled.append(new)
    return sorted(int(x) for x in disabled)


_ROWS, _COLS = 16384, 2048
_NW = 32
_RPW = _ROWS // _NW


_W = 128
_CHUNK = 256


@functools.cache
def _build(tof_count):
    disabled = _disabled_tofs(tof_count, 1, 3, 0.5)
    windows = sorted({(c // _W) * _W for c in disabled})
    nwin = len(windows)
    spans = []
    pos = 0
    for w in windows + [tof_count]:
        if w > pos:
            spans.append((pos, w - pos))
        pos = w + _W
    groups = {w: sorted({((c - w) // 16) * 16 for c in disabled
                         if (c // _W) * _W == w}) for w in windows}
    lanes = {w: {g: [c - w - g for c in disabled
                     if (c // _W) * _W == w and ((c - w) // 16) * 16 == g]
                 for g in groups[w]} for w in windows}
    nchunk = _RPW // _CHUNK
    mesh = plsc.VectorSubcoreMesh(core_axis_name="c", subcore_axis_name="s")

    @functools.partial(
        pl.kernel,
        mesh=mesh,
        out_type=jax.ShapeDtypeStruct((_ROWS, _COLS), jnp.float32),
        scratch_types=(
            [pltpu.VMEM((_CHUNK, _W), jnp.float32) for _ in range(nwin)]
            + [pltpu.SemaphoreType.DMA, pltpu.SemaphoreType.DMA]
        ),
    )
    def k(img_hbm, out_hbm, *rest):
        sbufs = rest[:nwin]
        span_sem, stripe_sem = rest[nwin], rest[nwin + 1]
        wid = lax.axis_index("s") * 2 + lax.axis_index("c")
        base = wid * _RPW
        rows = pl.ds(base, _RPW)
        iota = lax.iota(jnp.int32, 16)

        for c0, cw in spans:
            pltpu.make_async_copy(
                img_hbm.at[rows, pl.ds(c0, cw)],
                out_hbm.at[rows, pl.ds(c0, cw)], span_sem).start()

        def stripe_in(ci, w, sbuf):
            r = pl.ds(base + ci * _CHUNK, _CHUNK)
            return pltpu.make_async_copy(
                img_hbm.at[r, pl.ds(w, _W)], sbuf, stripe_sem)

        def stripe_out(ci, w, sbuf):
            r = pl.ds(base + ci * _CHUNK, _CHUNK)
            return pltpu.make_async_copy(
                sbuf, out_hbm.at[r, pl.ds(w, _W)], stripe_sem)

        for ci in range(nchunk):
            if ci > 0:
                for w, sbuf in zip(windows, sbufs):
                    stripe_out(ci - 1, w, sbuf).wait()
            for w, sbuf in zip(windows, sbufs):
                stripe_in(ci, w, sbuf).start()
            for w, sbuf in zip(windows, sbufs):
                stripe_in(ci, w, sbuf).wait()

            def fix(r, carry):
                for w, sbuf in zip(windows, sbufs):
                    for g in groups[w]:
                        v = sbuf[r, pl.ds(g, 16)]
                        keep = jnp.ones((16,), jnp.float32)
                        for lane in lanes[w][g]:
                            keep = jnp.where(iota == lane, 0.0, keep)
                        sbuf[r, pl.ds(g, 16)] = v * keep
                return carry

            lax.fori_loop(0, _CHUNK, fix, 0)
            for w, sbuf in zip(windows, sbufs):
                stripe_out(ci, w, sbuf).start()

        for w, sbuf in zip(windows, sbufs):
            stripe_out(nchunk - 1, w, sbuf).wait()
        for c0, cw in spans:
            pltpu.make_async_copy(
                img_hbm.at[rows, pl.ds(c0, cw)],
                out_hbm.at[rows, pl.ds(c0, cw)], span_sem).wait()

    return k


def kernel(img):
    return _build(img.shape[-1])(img)


# speedup vs baseline: 29.9234x; 29.9234x over previous
import functools

import jax
import jax.numpy as jnp
import numpy as np
from jax import lax
from jax.experimental import pallas as pl
from jax.experimental.pallas import tpu as pltpu
from jax.experimental.pallas import tpu_sc as plsc


def _disabled_tofs(tof_count, min_c, max_c, neighbor_p, seed=0):
    rng = np.random.default_rng(seed)
    count = int(rng.integers(min_c, max_c + 1))
    tof_list = rng.permutation(tof_count)
    first = int(rng.integers(1, tof_count))
    disabled = [first]
    tof_list = tof_list[tof_list != first]
    for _ in range(count - 1):
        r = float(rng.random())
        if r < neighbor_p:
            if r < neighbor_p / 2.0:
                offsets = (1, -1)
            else:
                offsets = (tof_count // 2, -(tof_count // 2))
            appended = False
            for d in list(disabled):
                for off in offsets:
                    cand = d + off
                    if cand in tof_list:
                        tof_list = tof_list[tof_list != cand]
                        disabled.append(int(cand))
                        appended = True
                        break
                if appended:
                    break
            if not appended:
                new = int(tof_list[0])
                tof_list = tof_list[tof_list != new]
                disabled.append(new)
        else:
            new = int(tof_list[0])
            tof_list = tof_list[tof_list != new]
            disabled.append(new)
    return sorted(int(x) for x in disabled)


_ROWS, _COLS = 16384, 2048
_NW = 32
_RPW = _ROWS // _NW
_R = 8
_NBUF = 4
_N = _RPW // _R


@functools.cache
def _build(tof_count):
    disabled = _disabled_tofs(tof_count, 1, 3, 0.5)
    mesh = plsc.VectorSubcoreMesh(core_axis_name="c", subcore_axis_name="s")

    @functools.partial(
        pl.kernel,
        mesh=mesh,
        out_type=jax.ShapeDtypeStruct((_ROWS, _COLS), jnp.float32),
        scratch_types=(
            [pltpu.VMEM((_R, _COLS), jnp.float32) for _ in range(_NBUF)]
            + [pltpu.SemaphoreType.DMA for _ in range(2 * _NBUF)]
        ),
    )
    def k(img_hbm, out_hbm, *rest):
        bufs = rest[:_NBUF]
        isems = rest[_NBUF:2 * _NBUF]
        osems = rest[2 * _NBUF:3 * _NBUF]
        wid = lax.axis_index("s") * 2 + lax.axis_index("c")
        base = wid * _RPW
        iota = lax.iota(jnp.int32, 16)

        def in_cp(i, b):
            r = pl.ds(base + i * _R, _R)
            return pltpu.make_async_copy(img_hbm.at[r, :], bufs[b], isems[b])

        def out_cp(i, b):
            r = pl.ds(base + i * _R, _R)
            return pltpu.make_async_copy(bufs[b], out_hbm.at[r, :], osems[b])

        in_cp(0, 0).start()
        in_cp(1, 1).start()

        def body(g, carry):
            for b in range(_NBUF):
                i = g * _NBUF + b
                j = i + 2
                bj = (b + 2) % _NBUF

                @pl.when(j < _N)
                def _():
                    @pl.when(j >= _NBUF)
                    def _():
                        out_cp(j - _NBUF, bj).wait()
                    in_cp(j, bj).start()

                in_cp(i, b).wait()
                for r in range(_R):
                    for c in disabled:
                        w = (c // 16) * 16
                        lane = c % 16
                        v = bufs[b][r, pl.ds(w, 16)]
                        bufs[b][r, pl.ds(w, 16)] = jnp.where(
                            iota == lane, 0.0, v)
                out_cp(i, b).start()
            return carry

        lax.fori_loop(0, _N // _NBUF, body, 0)
        for b in range(_NBUF):
            out_cp(_N - _NBUF + b, b).wait()

    return k


def kernel(img):
    return _build(img.shape[-1])(img)


# speedup vs baseline: 30.0050x; 1.0027x over previous
import functools

import jax
import jax.numpy as jnp
import numpy as np
from jax import lax
from jax.experimental import pallas as pl
from jax.experimental.pallas import tpu as pltpu
from jax.experimental.pallas import tpu_sc as plsc


def _disabled_tofs(tof_count, min_c, max_c, neighbor_p, seed=0):
    rng = np.random.default_rng(seed)
    count = int(rng.integers(min_c, max_c + 1))
    tof_list = rng.permutation(tof_count)
    first = int(rng.integers(1, tof_count))
    disabled = [first]
    tof_list = tof_list[tof_list != first]
    for _ in range(count - 1):
        r = float(rng.random())
        if r < neighbor_p:
            if r < neighbor_p / 2.0:
                offsets = (1, -1)
            else:
                offsets = (tof_count // 2, -(tof_count // 2))
            appended = False
            for d in list(disabled):
                for off in offsets:
                    cand = d + off
                    if cand in tof_list:
                        tof_list = tof_list[tof_list != cand]
                        disabled.append(int(cand))
                        appended = True
                        break
                if appended:
                    break
            if not appended:
                new = int(tof_list[0])
                tof_list = tof_list[tof_list != new]
                disabled.append(new)
        else:
            new = int(tof_list[0])
            tof_list = tof_list[tof_list != new]
            disabled.append(new)
    return sorted(int(x) for x in disabled)


_ROWS, _COLS = 16384, 2048
_NW = 32
_RPW = _ROWS // _NW
_R = 4
_NBUF = 8
_N = _RPW // _R


@functools.cache
def _build(tof_count):
    disabled = _disabled_tofs(tof_count, 1, 3, 0.5)
    mesh = plsc.VectorSubcoreMesh(core_axis_name="c", subcore_axis_name="s")

    @functools.partial(
        pl.kernel,
        mesh=mesh,
        out_type=jax.ShapeDtypeStruct((_ROWS, _COLS), jnp.float32),
        scratch_types=(
            [pltpu.VMEM((_R, _COLS), jnp.float32) for _ in range(_NBUF)]
            + [pltpu.SemaphoreType.DMA for _ in range(2 * _NBUF)]
        ),
    )
    def k(img_hbm, out_hbm, *rest):
        bufs = rest[:_NBUF]
        isems = rest[_NBUF:2 * _NBUF]
        osems = rest[2 * _NBUF:3 * _NBUF]
        wid = lax.axis_index("s") * 2 + lax.axis_index("c")
        base = wid * _RPW
        iota = lax.iota(jnp.int32, 16)

        def in_cp(i, b):
            r = pl.ds(base + i * _R, _R)
            return pltpu.make_async_copy(img_hbm.at[r, :], bufs[b], isems[b])

        def out_cp(i, b):
            r = pl.ds(base + i * _R, _R)
            return pltpu.make_async_copy(bufs[b], out_hbm.at[r, :], osems[b])

        for p in range(6):
            in_cp(p, p).start()

        def body(g, carry):
            for b in range(_NBUF):
                i = g * _NBUF + b
                j = i + 6
                bj = (b + 6) % _NBUF

                @pl.when(j < _N)
                def _():
                    @pl.when(j >= _NBUF)
                    def _():
                        out_cp(j - _NBUF, bj).wait()
                    in_cp(j, bj).start()

                in_cp(i, b).wait()
                for r in range(_R):
                    for c in disabled:
                        w = (c // 16) * 16
                        lane = c % 16
                        v = bufs[b][r, pl.ds(w, 16)]
                        bufs[b][r, pl.ds(w, 16)] = jnp.where(
                            iota == lane, 0.0, v)
                out_cp(i, b).start()
            return carry

        lax.fori_loop(0, _N // _NBUF, body, 0)
        for b in range(_NBUF):
            out_cp(_N - _NBUF + b, b).wait()

    return k


def kernel(img):
    return _build(img.shape[-1])(img)


# speedup vs baseline: 30.2836x; 1.0093x over previous
import functools

import jax
import jax.numpy as jnp
import numpy as np
from jax import lax
from jax.experimental import pallas as pl
from jax.experimental.pallas import tpu as pltpu
from jax.experimental.pallas import tpu_sc as plsc


def _disabled_tofs(tof_count, min_c, max_c, neighbor_p, seed=0):
    rng = np.random.default_rng(seed)
    count = int(rng.integers(min_c, max_c + 1))
    tof_list = rng.permutation(tof_count)
    first = int(rng.integers(1, tof_count))
    disabled = [first]
    tof_list = tof_list[tof_list != first]
    for _ in range(count - 1):
        r = float(rng.random())
        if r < neighbor_p:
            if r < neighbor_p / 2.0:
                offsets = (1, -1)
            else:
                offsets = (tof_count // 2, -(tof_count // 2))
            appended = False
            for d in list(disabled):
                for off in offsets:
                    cand = d + off
                    if cand in tof_list:
                        tof_list = tof_list[tof_list != cand]
                        disabled.append(int(cand))
                        appended = True
                        break
                if appended:
                    break
            if not appended:
                new = int(tof_list[0])
                tof_list = tof_list[tof_list != new]
                disabled.append(new)
        else:
            new = int(tof_list[0])
            tof_list = tof_list[tof_list != new]
            disabled.append(new)
    return sorted(int(x) for x in disabled)


_ROWS, _COLS = 16384, 2048
_NW = 32
_RPW = _ROWS // _NW
_R = 4
_NBUF = 8
_N = _RPW // _R


@functools.cache
def _build(tof_count):
    disabled = _disabled_tofs(tof_count, 1, 3, 0.5)
    mesh = plsc.VectorSubcoreMesh(core_axis_name="c", subcore_axis_name="s")

    @functools.partial(
        pl.kernel,
        mesh=mesh,
        out_type=jax.ShapeDtypeStruct((_ROWS, _COLS), jnp.float32),
        scratch_types=(
            [pltpu.VMEM((_R, _COLS), jnp.float32) for _ in range(_NBUF)]
            + [pltpu.SemaphoreType.DMA for _ in range(2 * _NBUF)]
        ),
    )
    def k(img_hbm, out_hbm, *rest):
        bufs = rest[:_NBUF]
        isems = rest[_NBUF:2 * _NBUF]
        osems = rest[2 * _NBUF:3 * _NBUF]
        wid = lax.axis_index("s") * 2 + lax.axis_index("c")
        base = wid * _RPW
        iota = lax.iota(jnp.int32, 16)

        def in_cp(i, b):
            r = pl.ds((i * _NW + wid) * _R, _R)
            return pltpu.make_async_copy(img_hbm.at[r, :], bufs[b], isems[b])

        def out_cp(i, b):
            r = pl.ds((i * _NW + wid) * _R, _R)
            return pltpu.make_async_copy(bufs[b], out_hbm.at[r, :], osems[b])

        for p in range(6):
            in_cp(p, p).start()

        def body(g, carry):
            for b in range(_NBUF):
                i = g * _NBUF + b
                j = i + 6
                bj = (b + 6) % _NBUF

                @pl.when(j < _N)
                def _():
                    @pl.when(j >= _NBUF)
                    def _():
                        out_cp(j - _NBUF, bj).wait()
                    in_cp(j, bj).start()

                in_cp(i, b).wait()
                for r in range(_R):
                    for c in disabled:
                        w = (c // 16) * 16
                        lane = c % 16
                        v = bufs[b][r, pl.ds(w, 16)]
                        bufs[b][r, pl.ds(w, 16)] = jnp.where(
                            iota == lane, 0.0, v)
                out_cp(i, b).start()
            return carry

        lax.fori_loop(0, _N // _NBUF, body, 0)
        for b in range(_NBUF):
            out_cp(_N - _NBUF + b, b).wait()

    return k


def kernel(img):
    return _build(img.shape[-1])(img)
